# baseline (speedup 1.0000x reference)
import functools

import jax
import jax.numpy as jnp
from jax import lax
from jax.experimental import pallas as pl
from jax.experimental.pallas import tpu as pltpu
from jax.experimental.pallas import tpu_sc as plsc

R = 256
P = 196
C = 256
NLEV = 4
STRIDES = (4.0, 8.0, 16.0, 32.0)
WS = (128, 64, 32, 16)
LEV_OFF = (0, 32768, 40960, 43008)
TOTAL_ROWS = 43520
CH = 28
NCH = 7
NW = 32
RPW = R // NW


def _floor_i32(x):
    xi = x.astype(jnp.int32)
    xf = xi.astype(jnp.float32)
    return xi - (xf > x).astype(jnp.int32)


def _sc_body(table, pxh, pyh, rch, out,
             rcols_v, px_v, py_v, idx_v, w_v, rows_v, outp_v,
             g0, g1, w0, w1):
    wsems = (w0, w1)
    iota = lax.iota(jnp.int32, 16)
    wid = lax.axis_index("s") * 2 + lax.axis_index("c")
    r0 = wid * RPW

    pltpu.sync_copy(rch, rcols_v)

    def _bcast_roi(j, r):
        return plsc.load_gather(rcols_v, [jnp.full((16,), j, jnp.int32),
                                          jnp.full((16,), r, jnp.int32)])

    def phase_a(r):
        pltpu.sync_copy(pxh.at[r], px_v)
        pltpu.sync_copy(pyh.at[r], py_v)
        vb = _bcast_roi(0, r).astype(jnp.int32)
        vb = jnp.clip(vb, 0, 1)
        vx1 = _bcast_roi(1, r)
        vy1 = _bcast_roi(2, r)
        vw = _bcast_roi(3, r) - vx1
        vh = _bcast_roi(4, r) - vy1

        def make_pa_body(g):
          def pa_body(chunk, carry):
            off = chunk * CH + g * 12
            pxv = plsc.load_gather(px_v, [iota + off])
            pyv = plsc.load_gather(py_v, [iota + off])
            cx = pxv * vw + vx1
            cy = pyv * vh + vy1
            for lev in range(NLEV):
                w_lev = WS[lev]
                inv = 1.0 / STRIDES[lev]
                x = cx * inv - 0.5
                y = cy * inv - 0.5
                x0 = _floor_i32(x)
                y0 = _floor_i32(y)
                fx = x - x0.astype(jnp.float32)
                fy = y - y0.astype(jnp.float32)
                x1i = x0 + 1
                y1i = y0 + 1
                vx0 = (x0 >= 0) & (x0 <= w_lev - 1)
                vx1ok = (x1i >= 0) & (x1i <= w_lev - 1)
                vy0 = (y0 >= 0) & (y0 <= w_lev - 1)
                vy1ok = (y1i >= 0) & (y1i <= w_lev - 1)
                wx0 = jnp.where(vx0, 1.0 - fx, 0.0)
                wx1 = jnp.where(vx1ok, fx, 0.0)
                wy0 = jnp.where(vy0, 1.0 - fy, 0.0)
                wy1 = jnp.where(vy1ok, fy, 0.0)
                xc0 = jnp.clip(x0, 0, w_lev - 1)
                xc1 = jnp.clip(x1i, 0, w_lev - 1)
                yc0 = jnp.clip(y0, 0, w_lev - 1)
                yc1 = jnp.clip(y1i, 0, w_lev - 1)
                base = vb * (w_lev * w_lev) + LEV_OFF[lev]
                row_y0 = base + yc0 * w_lev
                row_y1 = base + yc1 * w_lev
                rows = (row_y0 + xc0, row_y0 + xc1, row_y1 + xc0, row_y1 + xc1)
                wts = (wx0 * wy0, wx1 * wy0, wx0 * wy1, wx1 * wy1)
                arow = jnp.full((16,), lev * NCH + chunk, jnp.int32)
                for k in range(4):
                    acol = iota + (k * CH + g * 12)
                    plsc.store_scatter(idx_v, [arow, acol], rows[k])
                    plsc.store_scatter(w_v, [arow, acol], wts[k])
            return carry
          return pa_body

        for g in range(2):
            lax.fori_loop(0, NCH, make_pa_body(g), 0)

    def start_gather(lev, ch, buf, sem):
        arow = lev * NCH + ch
        return pltpu.async_copy(table.at[idx_v.at[arow]], rows_v.at[buf], sem)

    def combine_chunk(lev, ch, buf):
        rows2 = rows_v.at[buf]
        obuf = outp_v.at[ch % 2]
        awrow = lev * NCH + ch

        @plsc.parallel_loop(0, CH, 1, unroll=2)
        def pbody(p):
            ws0 = w_v[awrow, pl.ds(0 * CH + p, 16)][0]
            ws1 = w_v[awrow, pl.ds(1 * CH + p, 16)][0]
            ws2 = w_v[awrow, pl.ds(2 * CH + p, 16)][0]
            ws3 = w_v[awrow, pl.ds(3 * CH + p, 16)][0]
            fmt = plsc.PackFormat.INTERLEAVED
            for j in range(8):
                sl = pl.ds(j * 32, 32)
                a0, b0 = plsc.unpack(rows2[0 * CH + p, sl], format=fmt)
                a1, b1 = plsc.unpack(rows2[1 * CH + p, sl], format=fmt)
                a2, b2 = plsc.unpack(rows2[2 * CH + p, sl], format=fmt)
                a3, b3 = plsc.unpack(rows2[3 * CH + p, sl], format=fmt)
                acca = a0 * ws0 + a1 * ws1 + a2 * ws2 + a3 * ws3
                accb = b0 * ws0 + b1 * ws1 + b2 * ws2 + b3 * ws3
                obuf[p, sl] = plsc.pack(acca, accb, format=fmt)

    def out_slice(r, lev, ch):
        return out.at[r, pl.ds(ch * CH, CH), pl.ds(lev * C, C)]

    def start_write(r, lev, ch):
        return pltpu.async_copy(outp_v.at[ch % 2], out_slice(r, lev, ch),
                                wsems[ch % 2])

    def drain_write(parity):
        pltpu.make_async_copy(outp_v.at[parity], out_slice(r0, 0, parity),
                              wsems[parity]).wait()

    def task_body(t, carry):
        r = r0 + t // NLEV
        lev = t % NLEV

        @pl.when(lev == 0)
        def _():
            phase_a(r)

        gsems = (g0, g1)
        cur = start_gather(lev, 0, 0, g0)
        wh = [None, None]
        for ch in range(NCH):
            if ch + 1 < NCH:
                nxt = start_gather(lev, ch + 1, (ch + 1) % 2, gsems[(ch + 1) % 2])
            cur.wait()
            if ch < 2:
                @pl.when(t > 0)
                def _():
                    drain_write(ch % 2)
            else:
                wh[ch % 2].wait()
            combine_chunk(lev, ch, ch % 2)
            wh[ch % 2] = start_write(r, lev, ch)
            if ch + 1 < NCH:
                cur = nxt
        return carry

    lax.fori_loop(0, RPW * NLEV, task_body, 0)
    drain_write(0)
    drain_write(1)


@jax.jit
def kernel(feat0, feat1, feat2, feat3, rois, points):
    feats = (feat0, feat1, feat2, feat3)
    table = jnp.concatenate(
        [jnp.transpose(f, (0, 2, 3, 1)).reshape(-1, C) for f in feats],
        axis=0).astype(jnp.bfloat16)
    pxh = points[:, :, 0]
    pyh = points[:, :, 1]
    rch = jnp.transpose(rois)

    mesh = plsc.VectorSubcoreMesh(core_axis_name="c", subcore_axis_name="s")
    run = functools.partial(
        pl.kernel,
        out_type=jax.ShapeDtypeStruct((R, P, NLEV * C), jnp.bfloat16),
        mesh=mesh,
        compiler_params=pltpu.CompilerParams(
            use_tc_tiling_on_sc=False, needs_layout_passes=False),
        scratch_types=[
            pltpu.VMEM((5, R), jnp.float32),
            pltpu.VMEM((P,), jnp.float32),
            pltpu.VMEM((P,), jnp.float32),
            pltpu.VMEM((NLEV * NCH, 4 * CH), jnp.int32),
            pltpu.VMEM((NLEV * NCH, 128), jnp.float32),
            pltpu.VMEM((2, 4 * CH, C), jnp.bfloat16),
            pltpu.VMEM((2, CH, C), jnp.bfloat16),
            pltpu.SemaphoreType.DMA,
            pltpu.SemaphoreType.DMA,
            pltpu.SemaphoreType.DMA,
            pltpu.SemaphoreType.DMA,
        ],
    )(_sc_body)
    out_pm = run(table, pxh, pyh, rch)
    return jnp.transpose(out_pm, (0, 2, 1)).astype(jnp.float32)


# speedup vs baseline: 1.1126x; 1.1126x over previous
import functools

import jax
import jax.numpy as jnp
from jax import lax
from jax.experimental import pallas as pl
from jax.experimental.pallas import tpu as pltpu
from jax.experimental.pallas import tpu_sc as plsc

R = 256
P = 196
C = 256
NLEV = 4
STRIDES = (4.0, 8.0, 16.0, 32.0)
WS = (128, 64, 32, 16)
LEV_OFF = (0, 32768, 40960, 43008)
TOTAL_ROWS = 43520
CH = 28
NCH = 7
NW = 32
RPW = R // NW


def _floor_i32(x):
    xi = x.astype(jnp.int32)
    xf = xi.astype(jnp.float32)
    return xi - (xf > x).astype(jnp.int32)


def _sc_body(table, pxh, pyh, rch, out,
             rcols_v, px_v, py_v, idx_v, w_v, rows_v, outp_v,
             g0, g1, w0, w1):
    wsems = (w0, w1)
    iota = lax.iota(jnp.int32, 16)
    wid = lax.axis_index("s") * 2 + lax.axis_index("c")
    r0 = wid * RPW

    pltpu.sync_copy(rch, rcols_v)

    def _bcast_roi(j, r):
        return plsc.load_gather(rcols_v, [jnp.full((16,), j, jnp.int32),
                                          jnp.full((16,), r, jnp.int32)])

    def phase_a(r):
        pltpu.sync_copy(pxh.at[r], px_v)
        pltpu.sync_copy(pyh.at[r], py_v)
        vb = _bcast_roi(0, r).astype(jnp.int32)
        vb = jnp.clip(vb, 0, 1)
        vx1 = _bcast_roi(1, r)
        vy1 = _bcast_roi(2, r)
        vw = _bcast_roi(3, r) - vx1
        vh = _bcast_roi(4, r) - vy1

        def make_pa_body(g):
          def pa_body(chunk, carry):
            off = chunk * CH + g * 12
            pxv = plsc.load_gather(px_v, [iota + off])
            pyv = plsc.load_gather(py_v, [iota + off])
            cx = pxv * vw + vx1
            cy = pyv * vh + vy1
            for lev in range(NLEV):
                w_lev = WS[lev]
                inv = 1.0 / STRIDES[lev]
                x = cx * inv - 0.5
                y = cy * inv - 0.5
                x0 = _floor_i32(x)
                y0 = _floor_i32(y)
                fx = x - x0.astype(jnp.float32)
                fy = y - y0.astype(jnp.float32)
                x1i = x0 + 1
                y1i = y0 + 1
                vx0 = (x0 >= 0) & (x0 <= w_lev - 1)
                vx1ok = (x1i >= 0) & (x1i <= w_lev - 1)
                vy0 = (y0 >= 0) & (y0 <= w_lev - 1)
                vy1ok = (y1i >= 0) & (y1i <= w_lev - 1)
                wx0 = jnp.where(vx0, 1.0 - fx, 0.0)
                wx1 = jnp.where(vx1ok, fx, 0.0)
                wy0 = jnp.where(vy0, 1.0 - fy, 0.0)
                wy1 = jnp.where(vy1ok, fy, 0.0)
                xc0 = jnp.clip(x0, 0, w_lev - 1)
                xc1 = jnp.clip(x1i, 0, w_lev - 1)
                yc0 = jnp.clip(y0, 0, w_lev - 1)
                yc1 = jnp.clip(y1i, 0, w_lev - 1)
                base = vb * (w_lev * w_lev) + LEV_OFF[lev]
                row_y0 = base + yc0 * w_lev
                row_y1 = base + yc1 * w_lev
                rows = (row_y0 + xc0, row_y0 + xc1, row_y1 + xc0, row_y1 + xc1)
                wts = (wx0 * wy0, wx1 * wy0, wx0 * wy1, wx1 * wy1)
                arow = jnp.full((16,), lev * NCH + chunk, jnp.int32)
                for k in range(4):
                    acol = iota + (k * CH + g * 12)
                    plsc.store_scatter(idx_v, [arow, acol], rows[k])
                    plsc.store_scatter(w_v, [arow, acol], wts[k])
            return carry
          return pa_body

        for g in range(2):
            lax.fori_loop(0, NCH, make_pa_body(g), 0)

    def start_gather(lev, ch, buf, sem):
        arow = lev * NCH + ch
        return pltpu.async_copy(table.at[idx_v.at[arow]], rows_v.at[buf], sem)

    def combine_chunk(lev, ch, buf):
        rows2 = rows_v.at[buf]
        obuf = outp_v.at[ch % 2]
        awrow = lev * NCH + ch

        @plsc.parallel_loop(0, CH, 1, unroll=2)
        def pbody(p):
            ws0 = w_v[awrow, pl.ds(0 * CH + p, 16)][0]
            ws1 = w_v[awrow, pl.ds(1 * CH + p, 16)][0]
            ws2 = w_v[awrow, pl.ds(2 * CH + p, 16)][0]
            ws3 = w_v[awrow, pl.ds(3 * CH + p, 16)][0]
            for j in range(16):
                sl = pl.ds(j * 16, 16)
                acc = (rows2[0 * CH + p, sl] * ws0
                       + rows2[1 * CH + p, sl] * ws1
                       + rows2[2 * CH + p, sl] * ws2
                       + rows2[3 * CH + p, sl] * ws3)
                obuf[p, sl] = acc

    def out_slice(r, lev, ch):
        return out.at[r, pl.ds(ch * CH, CH), pl.ds(lev * C, C)]

    def start_write(r, lev, ch):
        return pltpu.async_copy(outp_v.at[ch % 2], out_slice(r, lev, ch),
                                wsems[ch % 2])

    def drain_write(parity):
        pltpu.make_async_copy(outp_v.at[parity], out_slice(r0, 0, parity),
                              wsems[parity]).wait()

    def task_body(t, carry):
        r = r0 + t // NLEV
        lev = t % NLEV

        @pl.when(lev == 0)
        def _():
            phase_a(r)

        gsems = (g0, g1)
        cur = start_gather(lev, 0, 0, g0)
        wh = [None, None]
        for ch in range(NCH):
            if ch + 1 < NCH:
                nxt = start_gather(lev, ch + 1, (ch + 1) % 2, gsems[(ch + 1) % 2])
            cur.wait()
            if ch < 2:
                @pl.when(t > 0)
                def _():
                    drain_write(ch % 2)
            else:
                wh[ch % 2].wait()
            combine_chunk(lev, ch, ch % 2)
            wh[ch % 2] = start_write(r, lev, ch)
            if ch + 1 < NCH:
                cur = nxt
        return carry

    lax.fori_loop(0, RPW * NLEV, task_body, 0)
    drain_write(0)
    drain_write(1)


@jax.jit
def kernel(feat0, feat1, feat2, feat3, rois, points):
    feats = (feat0, feat1, feat2, feat3)
    table = jnp.concatenate(
        [jnp.transpose(f, (0, 2, 3, 1)).reshape(-1, C) for f in feats], axis=0)
    pxh = points[:, :, 0]
    pyh = points[:, :, 1]
    rch = jnp.transpose(rois)

    mesh = plsc.VectorSubcoreMesh(core_axis_name="c", subcore_axis_name="s")
    run = functools.partial(
        pl.kernel,
        out_type=jax.ShapeDtypeStruct((R, P, NLEV * C), jnp.float32),
        mesh=mesh,
        compiler_params=pltpu.CompilerParams(
            use_tc_tiling_on_sc=False, needs_layout_passes=False),
        scratch_types=[
            pltpu.VMEM((5, R), jnp.float32),
            pltpu.VMEM((P,), jnp.float32),
            pltpu.VMEM((P,), jnp.float32),
            pltpu.VMEM((NLEV * NCH, 4 * CH), jnp.int32),
            pltpu.VMEM((NLEV * NCH, 128), jnp.float32),
            pltpu.VMEM((2, 4 * CH, C), jnp.float32),
            pltpu.VMEM((2, CH, C), jnp.float32),
            pltpu.SemaphoreType.DMA,
            pltpu.SemaphoreType.DMA,
            pltpu.SemaphoreType.DMA,
            pltpu.SemaphoreType.DMA,
        ],
    )(_sc_body)
    out_pm = run(table, pxh, pyh, rch)
    return jnp.transpose(out_pm, (0, 2, 1))
